# baseline (speedup 1.0000x reference)
import functools

import jax
import jax.numpy as jnp
from jax.experimental import pallas as pl
from jax.experimental.pallas import tpu as pltpu

N = 4096
IN_FEAT = 64
OUT_FEAT = 64
BM = 256


def _fused_kernel(feat_ref, w_ref, adj_ref, out_ref, x_ref):
    @pl.when(pl.program_id(0) == 0)
    def _():
        x_ref[...] = jnp.dot(
            feat_ref[...], w_ref[...], preferred_element_type=jnp.float32
        )

    out_ref[...] = jnp.dot(
        adj_ref[...], x_ref[...], preferred_element_type=jnp.float32
    )


@jax.jit
def kernel(feat, adj, weight2):
    grid = (N // BM,)
    return pl.pallas_call(
        _fused_kernel,
        grid=grid,
        in_specs=[
            pl.BlockSpec((N, IN_FEAT), lambda i: (0, 0)),
            pl.BlockSpec((IN_FEAT, OUT_FEAT), lambda i: (0, 0)),
            pl.BlockSpec((BM, N), lambda i: (i, 0)),
        ],
        out_specs=pl.BlockSpec((BM, OUT_FEAT), lambda i: (i, 0)),
        out_shape=jax.ShapeDtypeStruct((N, OUT_FEAT), jnp.float32),
        scratch_shapes=[pltpu.VMEM((N, OUT_FEAT), jnp.float32)],
    )(feat, weight2, adj)


# speedup vs baseline: 1.1194x; 1.1194x over previous
import functools

import jax
import jax.numpy as jnp
from jax.experimental import pallas as pl
from jax.experimental.pallas import tpu as pltpu

N = 4096
IN_FEAT = 64
OUT_FEAT = 64
BM = 512


def _fused_kernel(feat_ref, w_ref, adj_ref, out_ref, x_ref):
    @pl.when(pl.program_id(0) == 0)
    def _():
        x_ref[...] = jnp.dot(
            feat_ref[...], w_ref[...], preferred_element_type=jnp.float32
        )

    out_ref[...] = jnp.dot(
        adj_ref[...], x_ref[...], preferred_element_type=jnp.float32
    )


@jax.jit
def kernel(feat, adj, weight2):
    grid = (N // BM,)
    return pl.pallas_call(
        _fused_kernel,
        grid=grid,
        in_specs=[
            pl.BlockSpec((N, IN_FEAT), lambda i: (0, 0)),
            pl.BlockSpec((IN_FEAT, OUT_FEAT), lambda i: (0, 0)),
            pl.BlockSpec((BM, N), lambda i: (i, 0)),
        ],
        out_specs=pl.BlockSpec((BM, OUT_FEAT), lambda i: (i, 0)),
        out_shape=jax.ShapeDtypeStruct((N, OUT_FEAT), jnp.float32),
        scratch_shapes=[pltpu.VMEM((N, OUT_FEAT), jnp.float32)],
    )(feat, weight2, adj)
